# Initial kernel scaffold; baseline (speedup 1.0000x reference)
import jax
import jax.numpy as jnp
from jax.experimental import pallas as pl


def kernel(x, table, pos_enc):
    raise NotImplementedError("write your pallas kernel here")


# speedup vs baseline: 2.8672x; 2.8672x over previous
import functools

import jax
import jax.numpy as jnp
from jax import lax
from jax.experimental import pallas as pl
from jax.experimental.pallas import tpu as pltpu
from jax.experimental.pallas import tpu_sc as plsc

NUM_EMBEDDINGS = 100000
D = 64
SEQ = 200
B = 4096
NC = 2
NS = 16
NW = NC * NS
SEQ_PER_W = B // NW
HALF = SEQ // 2
CHUNKS = B * SEQ // HALF
LANES = 16


def _emb_body(x_hbm, table_hbm, pos_hbm, out_hbm, pos_v, idx_v, rows_v, sem):
    wid = lax.axis_index("s") * NC + lax.axis_index("c")
    pltpu.sync_copy(pos_hbm, pos_v)
    seq0 = wid * SEQ_PER_W

    def seq_body(j, carry):
        seq = seq0 + j
        pltpu.sync_copy(x_hbm.at[seq], idx_v)
        pltpu.async_copy(table_hbm.at[idx_v.at[0]], rows_v.at[pl.ds(0, HALF)], sem).wait()
        pltpu.async_copy(table_hbm.at[idx_v.at[1]], rows_v.at[pl.ds(HALF, HALF)], sem).wait()

        def add_body(r, c2):
            for c in range(D // LANES):
                sl = pl.ds(c * LANES, LANES)
                rows_v[r, sl] = rows_v[r, sl] + pos_v[r, sl]
            return c2

        lax.fori_loop(0, SEQ, add_body, 0)
        pltpu.sync_copy(rows_v, out_hbm.at[pl.ds(seq * SEQ, SEQ)])
        return carry

    lax.fori_loop(0, SEQ_PER_W, seq_body, 0)


@jax.jit
def _emb(x_flat, table, pos_enc):
    mesh = plsc.VectorSubcoreMesh(core_axis_name="c", subcore_axis_name="s")
    f = functools.partial(
        pl.kernel,
        mesh=mesh,
        out_type=jax.ShapeDtypeStruct((B * SEQ, D), jnp.float32),
        scratch_types=[
            pltpu.VMEM((SEQ, D), jnp.float32),
            pltpu.VMEM((2, HALF), jnp.int32),
            pltpu.VMEM((SEQ, D), jnp.float32),
            pltpu.SemaphoreType.DMA,
        ],
        compiler_params=pltpu.CompilerParams(use_tc_tiling_on_sc=False),
    )(_emb_body)
    return f(x_flat, table, pos_enc)


def kernel(x, table, pos_enc):
    x_flat = x.astype(jnp.int32).reshape(B, 2, HALF)
    out = _emb(x_flat, table, pos_enc)
    return out.reshape(B, SEQ, D)


# speedup vs baseline: 4.0809x; 1.4233x over previous
import functools

import jax
import jax.numpy as jnp
from jax import lax
from jax.experimental import pallas as pl
from jax.experimental.pallas import tpu as pltpu
from jax.experimental.pallas import tpu_sc as plsc

NUM_EMBEDDINGS = 100000
D = 64
SEQ = 200
B = 4096
NC = 2
NS = 16
NW = NC * NS
CPW = B // NW
HALF = SEQ // 2
LANES = 16
NB = 4
GROUPS = CPW // NB


def _emb_body(x_hbm, table_hbm, pos_hbm, out_hbm, pos_v,
              i0, i1, i2, i3, r0, r1, r2, r3,
              is0, is1, is2, is3, gs0, gs1, gs2, gs3,
              ss0, ss1, ss2, ss3):
    idx = (i0, i1, i2, i3)
    rows = (r0, r1, r2, r3)
    isem = (is0, is1, is2, is3)
    gsem = (gs0, gs1, gs2, gs3)
    ssem = (ss0, ss1, ss2, ss3)

    wid = lax.axis_index("s") * NC + lax.axis_index("c")
    pltpu.sync_copy(pos_hbm, pos_v)
    c0 = wid * CPW

    def issue_idx(cg, b):
        pltpu.async_copy(x_hbm.at[cg], idx[b], isem[b])

    def wait_idx(cg, b):
        pltpu.make_async_copy(x_hbm.at[cg], idx[b], isem[b]).wait()

    def issue_gather(b):
        for k in range(2):
            pltpu.async_copy(table_hbm.at[idx[b].at[k]],
                             rows[b].at[pl.ds(k * HALF, HALF)], gsem[b])

    def wait_gather(b):
        for k in range(2):
            pltpu.make_async_copy(table_hbm.at[idx[b].at[k]],
                                  rows[b].at[pl.ds(k * HALF, HALF)],
                                  gsem[b]).wait()

    def issue_store(cg, b):
        pltpu.async_copy(rows[b], out_hbm.at[pl.ds(cg * SEQ, SEQ)], ssem[b])

    def wait_store(cg, b):
        pltpu.make_async_copy(rows[b], out_hbm.at[pl.ds(cg * SEQ, SEQ)],
                              ssem[b]).wait()

    def add(b):
        def add_body(r, carry):
            for col in range(D // LANES):
                sl = pl.ds(col * LANES, LANES)
                rows[b][r, sl] = rows[b][r, sl] + pos_v[r, sl]
            return carry
        lax.fori_loop(0, SEQ, add_body, 0)

    def stage(c, b, wait_prev_store=True, issue_next=True):
        cg = c0 + c
        wait_gather(b)
        b2 = (b + 2) % NB
        if issue_next:
            issue_idx(cg + 2, b2)
        add(b)
        issue_store(cg, b)
        if issue_next:
            wait_idx(cg + 2, b2)
            if wait_prev_store:
                wait_store(cg - 2, b2)
            issue_gather(b2)

    issue_idx(c0 + 0, 0)
    issue_idx(c0 + 1, 1)
    wait_idx(c0 + 0, 0)
    issue_gather(0)
    wait_idx(c0 + 1, 1)
    issue_gather(1)

    stage(0, 0, wait_prev_store=False)
    stage(1, 1, wait_prev_store=False)
    stage(2, 2)
    stage(3, 3)

    def group_body(g, carry):
        for b in range(NB):
            stage(g * NB + b, b)
        return carry
    lax.fori_loop(1, GROUPS - 1, group_body, 0)

    last = (GROUPS - 1) * NB
    stage(last + 0, 0)
    stage(last + 1, 1)
    stage(last + 2, 2, issue_next=False)
    stage(last + 3, 3, issue_next=False)

    for b in range(NB):
        wait_store(c0 + last + b, b)


@jax.jit
def _emb(x_flat, table, pos_enc):
    mesh = plsc.VectorSubcoreMesh(core_axis_name="c", subcore_axis_name="s")
    f = functools.partial(
        pl.kernel,
        mesh=mesh,
        out_type=jax.ShapeDtypeStruct((B * SEQ, D), jnp.float32),
        scratch_types=(
            [pltpu.VMEM((SEQ, D), jnp.float32)]
            + [pltpu.VMEM((2, HALF), jnp.int32) for _ in range(NB)]
            + [pltpu.VMEM((SEQ, D), jnp.float32) for _ in range(NB)]
            + [pltpu.SemaphoreType.DMA for _ in range(3 * NB)]
        ),
        compiler_params=pltpu.CompilerParams(use_tc_tiling_on_sc=False),
    )(_emb_body)
    return f(x_flat, table, pos_enc)


def kernel(x, table, pos_enc):
    x_flat = x.astype(jnp.int32).reshape(B, 2, HALF)
    out = _emb(x_flat, table, pos_enc)
    return out.reshape(B, SEQ, D)


# speedup vs baseline: 4.1451x; 1.0157x over previous
import functools

import jax
import jax.numpy as jnp
from jax import lax
from jax.experimental import pallas as pl
from jax.experimental.pallas import tpu as pltpu
from jax.experimental.pallas import tpu_sc as plsc

NUM_EMBEDDINGS = 100000
D = 64
SEQ = 200
B = 4096
NC = 2
NS = 16
NW = NC * NS
CPW = B // NW
SPLITS = ((0, 128), (128, 72))
LANES = 16
NB = 4
GROUPS = CPW // NB


def _emb_body(x_hbm, table_hbm, pos_hbm, out_hbm, pos_v,
              i0, i1, i2, i3, r0, r1, r2, r3,
              is0, is1, is2, is3, gs0, gs1, gs2, gs3,
              ss0, ss1, ss2, ss3):
    idx = (i0, i1, i2, i3)
    rows = (r0, r1, r2, r3)
    isem = (is0, is1, is2, is3)
    gsem = (gs0, gs1, gs2, gs3)
    ssem = (ss0, ss1, ss2, ss3)

    wid = lax.axis_index("s") * NC + lax.axis_index("c")
    pltpu.sync_copy(pos_hbm, pos_v)
    c0 = wid * CPW

    def issue_idx(cg, b):
        pltpu.async_copy(x_hbm.at[cg], idx[b], isem[b])

    def wait_idx(cg, b):
        pltpu.make_async_copy(x_hbm.at[cg], idx[b], isem[b]).wait()

    def issue_gather(b):
        for off, n in SPLITS:
            pltpu.async_copy(table_hbm.at[idx[b].at[pl.ds(off, n)]],
                             rows[b].at[pl.ds(off, n)], gsem[b])

    def wait_gather(b):
        for off, n in SPLITS:
            pltpu.make_async_copy(table_hbm.at[idx[b].at[pl.ds(off, n)]],
                                  rows[b].at[pl.ds(off, n)],
                                  gsem[b]).wait()

    def issue_store(cg, b):
        pltpu.async_copy(rows[b], out_hbm.at[cg], ssem[b])

    def wait_store(cg, b):
        pltpu.make_async_copy(rows[b], out_hbm.at[cg], ssem[b]).wait()

    def add(b):
        def add_body(r, carry):
            for col in range(D // LANES):
                sl = pl.ds(col * LANES, LANES)
                rows[b][r, sl] = rows[b][r, sl] + pos_v[r, sl]
            return carry
        lax.fori_loop(0, SEQ, add_body, 0)

    def stage(c, b, wait_prev_store=True, issue_next=True):
        cg = c0 + c
        wait_gather(b)
        b2 = (b + 2) % NB
        if issue_next:
            issue_idx(cg + 2, b2)
        add(b)
        issue_store(cg, b)
        if issue_next:
            wait_idx(cg + 2, b2)
            if wait_prev_store:
                wait_store(cg - 2, b2)
            issue_gather(b2)

    issue_idx(c0 + 0, 0)
    issue_idx(c0 + 1, 1)
    wait_idx(c0 + 0, 0)
    issue_gather(0)
    wait_idx(c0 + 1, 1)
    issue_gather(1)

    stage(0, 0, wait_prev_store=False)
    stage(1, 1, wait_prev_store=False)
    stage(2, 2)
    stage(3, 3)

    def group_body(g, carry):
        for b in range(NB):
            stage(g * NB + b, b)
        return carry
    lax.fori_loop(1, GROUPS - 1, group_body, 0)

    last = (GROUPS - 1) * NB
    stage(last + 0, 0)
    stage(last + 1, 1)
    stage(last + 2, 2, issue_next=False)
    stage(last + 3, 3, issue_next=False)

    for b in range(NB):
        wait_store(c0 + last + b, b)


@jax.jit
def _emb(x_flat, table, pos_enc):
    mesh = plsc.VectorSubcoreMesh(core_axis_name="c", subcore_axis_name="s")
    f = functools.partial(
        pl.kernel,
        mesh=mesh,
        out_type=jax.ShapeDtypeStruct((B, SEQ, D), jnp.float32),
        scratch_types=(
            [pltpu.VMEM((SEQ, D), jnp.float32)]
            + [pltpu.VMEM((SEQ,), jnp.int32) for _ in range(NB)]
            + [pltpu.VMEM((SEQ, D), jnp.float32) for _ in range(NB)]
            + [pltpu.SemaphoreType.DMA for _ in range(3 * NB)]
        ),
        compiler_params=pltpu.CompilerParams(use_tc_tiling_on_sc=False),
    )(_emb_body)
    return f(x_flat, table, pos_enc)


def kernel(x, table, pos_enc):
    return _emb(x.astype(jnp.int32), table, pos_enc)
